# baseline (speedup 1.0000x reference)
import jax
import jax.numpy as jnp
from jax import lax
from jax.experimental import pallas as pl
from jax.experimental.pallas import tpu as pltpu

_N = 20000
_NP = 20480
_RN = 160
_PRE = 300
_CAP = 384
_DETS = 100
_BORDER = 2.0
_WIN = 1024.0
_IMG = 4096.0
_SAMP = 2.0
_CLS_OFF = 8192.0
_NMS_T = 0.5
_SCORE_T = 1e-8


def _nms_kernel(x1_ref, y1_ref, x2_ref, y2_ref, s_ref, c_ref, w_ref, out_ref,
                s_scr):
    x1 = x1_ref[...]
    y1 = y1_ref[...]
    x2 = x2_ref[...]
    y2 = y2_ref[...]
    s = s_ref[...]
    cls = c_ref[...]

    good = (x1 >= _BORDER) & (y1 >= _BORDER) \
         & (x2 < _WIN - _BORDER) & (y2 < _WIN - _BORDER)
    drone = (cls == 80.0) | (cls == 81.0)
    valid = good & drone & (s > _SCORE_T)
    s_scr[...] = jnp.where(valid, s, 0.0)

    flat = (lax.broadcasted_iota(jnp.int32, (_RN, 128), 0) * 128
            + lax.broadcasted_iota(jnp.int32, (_RN, 128), 1))
    lane = lax.broadcasted_iota(jnp.int32, (1, 128), 1)
    cap_iota = lax.broadcasted_iota(jnp.int32, (1, _CAP), 1)
    zrow = jnp.zeros((1, _CAP), jnp.float32)

    def sel_body(i, carry):
        ts, ux1, uy1, ux2, uy2, uc = carry
        s_work = s_scr[...]
        m = jnp.max(s_work)
        idx = jnp.min(jnp.where(s_work == m, flat, jnp.int32(_NP)))
        r = idx // 128
        c = idx - r * 128
        lm = (lane == c).astype(jnp.float32)
        vx1 = jnp.sum(x1_ref[pl.ds(r, 1), :] * lm)
        vy1 = jnp.sum(y1_ref[pl.ds(r, 1), :] * lm)
        vx2 = jnp.sum(x2_ref[pl.ds(r, 1), :] * lm)
        vy2 = jnp.sum(y2_ref[pl.ds(r, 1), :] * lm)
        vc = jnp.sum(c_ref[pl.ds(r, 1), :] * lm)
        vw = jnp.sum(w_ref[pl.ds(r, 1), :] * lm)
        wq = jnp.floor(vw * 0.25)
        offx = wq * 1024.0
        offy = (vw - wq * 4.0) * 1024.0
        hit = (cap_iota == i).astype(jnp.float32)
        ts = ts + hit * m
        ux1 = ux1 + hit * jnp.clip(vx1 * _SAMP + offx, 0.0, _IMG)
        uy1 = uy1 + hit * jnp.clip(vy1 * _SAMP + offy, 0.0, _IMG)
        ux2 = ux2 + hit * jnp.clip(vx2 * _SAMP + offx, 0.0, _IMG)
        uy2 = uy2 + hit * jnp.clip(vy2 * _SAMP + offy, 0.0, _IMG)
        uc = uc + hit * vc
        srow = s_scr[pl.ds(r, 1), :]
        s_scr[pl.ds(r, 1), :] = jnp.where(lane == c, -1.0, srow)
        return (ts, ux1, uy1, ux2, uy2, uc)

    carry0 = (zrow, zrow, zrow, zrow, zrow, zrow)
    ts, ux1, uy1, ux2, uy2, uc = lax.fori_loop(0, _PRE, sel_body, carry0)

    co = uc * _CLS_OFF
    rx1 = ux1 + co
    ry1 = uy1 + co
    rx2 = ux2 + co
    ry2 = uy2 + co
    area = (rx2 - rx1) * (ry2 - ry1)
    keep0 = (ts > 0.0).astype(jnp.float32)

    def nms_body(i, keep):
        hm = (cap_iota == i).astype(jnp.float32)
        bx1 = jnp.sum(hm * rx1)
        by1 = jnp.sum(hm * ry1)
        bx2 = jnp.sum(hm * rx2)
        by2 = jnp.sum(hm * ry2)
        ai = jnp.sum(hm * area)
        ki = jnp.sum(hm * keep) > 0.0
        wi = jnp.clip(jnp.minimum(bx2, rx2) - jnp.maximum(bx1, rx1), 0.0, None)
        hi = jnp.clip(jnp.minimum(by2, ry2) - jnp.maximum(by1, ry1), 0.0, None)
        inter = wi * hi
        iou = inter / (ai + area - inter + 1e-9)
        sup = (iou > _NMS_T) & (cap_iota > i) & ki
        return jnp.where(sup, 0.0, keep)

    keep = lax.fori_loop(0, _PRE, nms_body, keep0)
    fs = keep * ts

    row384 = lax.broadcasted_iota(jnp.int32, (_CAP, _CAP), 0)
    col384 = lax.broadcasted_iota(jnp.int32, (_CAP, _CAP), 1)
    ltri = (row384 <= col384).astype(jnp.float32)
    real = (cap_iota < _PRE).astype(jnp.float32)
    nk = (1.0 - keep) * real
    incl_k = jax.lax.dot_general(keep, ltri, (((1,), (0,)), ((), ())),
                                 precision=lax.Precision.HIGHEST,
                                 preferred_element_type=jnp.float32)
    incl_n = jax.lax.dot_general(nk, ltri, (((1,), (0,)), ((), ())),
                                 precision=lax.Precision.HIGHEST,
                                 preferred_element_type=jnp.float32)
    nkept = jnp.sum(keep)
    dest = jnp.where(keep > 0.0, incl_k - 1.0,
                     jnp.where(nk > 0.0, nkept + incl_n - 1.0, 9999.0))
    pt = (row384.astype(jnp.float32) == dest).astype(jnp.float32)
    data = jnp.concatenate(
        [ux1, uy1, ux2, uy2, fs, zrow, zrow, zrow], axis=0)
    outm = jax.lax.dot_general(data, pt, (((1,), (1,)), ((), ())),
                               precision=lax.Precision.HIGHEST,
                               preferred_element_type=jnp.float32)
    out_ref[...] = outm[:, 0:128]


def kernel(boxes, scores, classes, window_idx):
    pad = _NP - _N

    def prep(v):
        return jnp.pad(v, (0, pad)).reshape(_RN, 128)

    x1 = prep(boxes[:, 0])
    y1 = prep(boxes[:, 1])
    x2 = prep(boxes[:, 2])
    y2 = prep(boxes[:, 3])
    s = prep(scores)
    c = prep(classes.astype(jnp.float32))
    w = prep(window_idx.astype(jnp.float32))

    out = pl.pallas_call(
        _nms_kernel,
        out_shape=jax.ShapeDtypeStruct((8, 128), jnp.float32),
        scratch_shapes=[pltpu.VMEM((_RN, 128), jnp.float32)],
    )(x1, y1, x2, y2, s, c, w)

    return jnp.stack([out[0, :_DETS], out[1, :_DETS], out[2, :_DETS],
                      out[3, :_DETS], out[4, :_DETS]], axis=1)


# speedup vs baseline: 1.1136x; 1.1136x over previous
import jax
import jax.numpy as jnp
from jax import lax
from jax.experimental import pallas as pl
from jax.experimental.pallas import tpu as pltpu

_N = 20000
_NP = 20480
_RN = 160
_PRE = 300
_CAP = 384
_DETS = 100
_BORDER = 2.0
_WIN = 1024.0
_IMG = 4096.0
_SAMP = 2.0
_CLS_OFF = 8192.0
_NMS_T = 0.5
_SCORE_T = 1e-8


def _nms_kernel(x1_ref, y1_ref, x2_ref, y2_ref, s_ref, c_ref, w_ref, out_ref,
                s_scr):
    x1 = x1_ref[...]
    y1 = y1_ref[...]
    x2 = x2_ref[...]
    y2 = y2_ref[...]
    s = s_ref[...]
    cls = c_ref[...]

    good = (x1 >= _BORDER) & (y1 >= _BORDER) \
         & (x2 < _WIN - _BORDER) & (y2 < _WIN - _BORDER)
    drone = (cls == 80.0) | (cls == 81.0)
    valid = good & drone & (s > _SCORE_T)
    s_scr[...] = jnp.where(valid, s, 0.0)

    flat = (lax.broadcasted_iota(jnp.int32, (_RN, 128), 0) * 128
            + lax.broadcasted_iota(jnp.int32, (_RN, 128), 1))
    lane = lax.broadcasted_iota(jnp.int32, (1, 128), 1)
    cap_iota = lax.broadcasted_iota(jnp.int32, (1, _CAP), 1)
    zrow = jnp.zeros((1, _CAP), jnp.float32)

    def pick(s_work, slot, carry):
        ts, ux1, uy1, ux2, uy2, uc = carry
        m = jnp.max(s_work)
        idx = jnp.min(jnp.where(s_work == m, flat, jnp.int32(_NP)))
        r = idx // 128
        c = idx - r * 128
        lm = (lane == c).astype(jnp.float32)
        vx1 = jnp.sum(x1_ref[pl.ds(r, 1), :] * lm)
        vy1 = jnp.sum(y1_ref[pl.ds(r, 1), :] * lm)
        vx2 = jnp.sum(x2_ref[pl.ds(r, 1), :] * lm)
        vy2 = jnp.sum(y2_ref[pl.ds(r, 1), :] * lm)
        vc = jnp.sum(c_ref[pl.ds(r, 1), :] * lm)
        vw = jnp.sum(w_ref[pl.ds(r, 1), :] * lm)
        wq = jnp.floor(vw * 0.25)
        offx = wq * 1024.0
        offy = (vw - wq * 4.0) * 1024.0
        hit = (cap_iota == slot).astype(jnp.float32)
        ts = ts + hit * m
        ux1 = ux1 + hit * jnp.clip(vx1 * _SAMP + offx, 0.0, _IMG)
        uy1 = uy1 + hit * jnp.clip(vy1 * _SAMP + offy, 0.0, _IMG)
        ux2 = ux2 + hit * jnp.clip(vx2 * _SAMP + offx, 0.0, _IMG)
        uy2 = uy2 + hit * jnp.clip(vy2 * _SAMP + offy, 0.0, _IMG)
        uc = uc + hit * vc
        s_work = jnp.where(flat == idx, -1.0, s_work)
        return s_work, (ts, ux1, uy1, ux2, uy2, uc)

    def sel_body(i, carry):
        s_work = s_scr[...]
        s_work, carry = pick(s_work, 2 * i, carry)
        s_work, carry = pick(s_work, 2 * i + 1, carry)
        s_scr[...] = s_work
        return carry

    carry0 = (zrow, zrow, zrow, zrow, zrow, zrow)
    ts, ux1, uy1, ux2, uy2, uc = lax.fori_loop(0, _PRE // 2, sel_body, carry0)

    co = uc * _CLS_OFF
    rx1 = ux1 + co
    ry1 = uy1 + co
    rx2 = ux2 + co
    ry2 = uy2 + co
    area = (rx2 - rx1) * (ry2 - ry1)
    keep0 = (ts > 0.0).astype(jnp.float32)

    def nms_body(i, keep):
        hm = (cap_iota == i).astype(jnp.float32)
        bx1 = jnp.sum(hm * rx1)
        by1 = jnp.sum(hm * ry1)
        bx2 = jnp.sum(hm * rx2)
        by2 = jnp.sum(hm * ry2)
        ai = jnp.sum(hm * area)
        ki = jnp.sum(hm * keep) > 0.0
        wi = jnp.clip(jnp.minimum(bx2, rx2) - jnp.maximum(bx1, rx1), 0.0, None)
        hi = jnp.clip(jnp.minimum(by2, ry2) - jnp.maximum(by1, ry1), 0.0, None)
        inter = wi * hi
        iou = inter / (ai + area - inter + 1e-9)
        sup = (iou > _NMS_T) & (cap_iota > i) & ki
        return jnp.where(sup, 0.0, keep)

    keep = lax.fori_loop(0, _PRE, nms_body, keep0)
    fs = keep * ts

    row384 = lax.broadcasted_iota(jnp.int32, (_CAP, _CAP), 0)
    col384 = lax.broadcasted_iota(jnp.int32, (_CAP, _CAP), 1)
    ltri = (row384 <= col384).astype(jnp.float32)
    real = (cap_iota < _PRE).astype(jnp.float32)
    nk = (1.0 - keep) * real
    incl_k = jax.lax.dot_general(keep, ltri, (((1,), (0,)), ((), ())),
                                 precision=lax.Precision.HIGHEST,
                                 preferred_element_type=jnp.float32)
    incl_n = jax.lax.dot_general(nk, ltri, (((1,), (0,)), ((), ())),
                                 precision=lax.Precision.HIGHEST,
                                 preferred_element_type=jnp.float32)
    nkept = jnp.sum(keep)
    dest = jnp.where(keep > 0.0, incl_k - 1.0,
                     jnp.where(nk > 0.0, nkept + incl_n - 1.0, 9999.0))
    pt = (row384.astype(jnp.float32) == dest).astype(jnp.float32)
    data = jnp.concatenate(
        [ux1, uy1, ux2, uy2, fs, zrow, zrow, zrow], axis=0)
    outm = jax.lax.dot_general(data, pt, (((1,), (1,)), ((), ())),
                               precision=lax.Precision.HIGHEST,
                               preferred_element_type=jnp.float32)
    out_ref[...] = outm[:, 0:128]


def kernel(boxes, scores, classes, window_idx):
    pad = _NP - _N

    def prep(v):
        return jnp.pad(v, (0, pad)).reshape(_RN, 128)

    x1 = prep(boxes[:, 0])
    y1 = prep(boxes[:, 1])
    x2 = prep(boxes[:, 2])
    y2 = prep(boxes[:, 3])
    s = prep(scores)
    c = prep(classes.astype(jnp.float32))
    w = prep(window_idx.astype(jnp.float32))

    out = pl.pallas_call(
        _nms_kernel,
        out_shape=jax.ShapeDtypeStruct((8, 128), jnp.float32),
        scratch_shapes=[pltpu.VMEM((_RN, 128), jnp.float32)],
    )(x1, y1, x2, y2, s, c, w)

    return jnp.stack([out[0, :_DETS], out[1, :_DETS], out[2, :_DETS],
                      out[3, :_DETS], out[4, :_DETS]], axis=1)


# speedup vs baseline: 1.1986x; 1.0763x over previous
import jax
import jax.numpy as jnp
from jax import lax
from jax.experimental import pallas as pl
from jax.experimental.pallas import tpu as pltpu

_N = 20000
_NP = 20480
_RN = 160
_PRE = 300
_CAP = 384
_DETS = 100
_BORDER = 2.0
_WIN = 1024.0
_IMG = 4096.0
_SAMP = 2.0
_CLS_OFF = 8192.0
_NMS_T = 0.5
_SCORE_T = 1e-8


def _nms_kernel(x1_ref, y1_ref, x2_ref, y2_ref, s_ref, c_ref, w_ref, out_ref,
                s_scr):
    x1 = x1_ref[...]
    y1 = y1_ref[...]
    x2 = x2_ref[...]
    y2 = y2_ref[...]
    s = s_ref[...]
    cls = c_ref[...]

    good = (x1 >= _BORDER) & (y1 >= _BORDER) \
         & (x2 < _WIN - _BORDER) & (y2 < _WIN - _BORDER)
    drone = (cls == 80.0) | (cls == 81.0)
    valid = good & drone & (s > _SCORE_T)
    s_scr[...] = jnp.where(valid, s, 0.0)

    flat = (lax.broadcasted_iota(jnp.int32, (_RN, 128), 0) * 128
            + lax.broadcasted_iota(jnp.int32, (_RN, 128), 1))
    lane = lax.broadcasted_iota(jnp.int32, (1, 128), 1)
    cap_iota = lax.broadcasted_iota(jnp.int32, (1, _CAP), 1)
    zrow = jnp.zeros((1, _CAP), jnp.float32)

    def pick(s_work, slot, carry):
        ts, ux1, uy1, ux2, uy2, uc = carry
        m = jnp.max(s_work)
        idx = jnp.min(jnp.where(s_work == m, flat, jnp.int32(_NP)))
        r = idx // 128
        c = idx - r * 128
        lm = (lane == c).astype(jnp.float32)
        vx1 = jnp.sum(x1_ref[pl.ds(r, 1), :] * lm)
        vy1 = jnp.sum(y1_ref[pl.ds(r, 1), :] * lm)
        vx2 = jnp.sum(x2_ref[pl.ds(r, 1), :] * lm)
        vy2 = jnp.sum(y2_ref[pl.ds(r, 1), :] * lm)
        vc = jnp.sum(c_ref[pl.ds(r, 1), :] * lm)
        vw = jnp.sum(w_ref[pl.ds(r, 1), :] * lm)
        wq = jnp.floor(vw * 0.25)
        offx = wq * 1024.0
        offy = (vw - wq * 4.0) * 1024.0
        hit = (cap_iota == slot).astype(jnp.float32)
        ts = ts + hit * m
        ux1 = ux1 + hit * jnp.clip(vx1 * _SAMP + offx, 0.0, _IMG)
        uy1 = uy1 + hit * jnp.clip(vy1 * _SAMP + offy, 0.0, _IMG)
        ux2 = ux2 + hit * jnp.clip(vx2 * _SAMP + offx, 0.0, _IMG)
        uy2 = uy2 + hit * jnp.clip(vy2 * _SAMP + offy, 0.0, _IMG)
        uc = uc + hit * vc
        s_work = jnp.where(flat == idx, -1.0, s_work)
        return s_work, (ts, ux1, uy1, ux2, uy2, uc)

    def sel_body(i, carry):
        s_work = s_scr[...]
        s_work, carry = pick(s_work, 4 * i, carry)
        s_work, carry = pick(s_work, 4 * i + 1, carry)
        s_work, carry = pick(s_work, 4 * i + 2, carry)
        s_work, carry = pick(s_work, 4 * i + 3, carry)
        s_scr[...] = s_work
        return carry

    carry0 = (zrow, zrow, zrow, zrow, zrow, zrow)
    ts, ux1, uy1, ux2, uy2, uc = lax.fori_loop(0, _PRE // 4, sel_body, carry0)

    co = uc * _CLS_OFF
    rx1 = ux1 + co
    ry1 = uy1 + co
    rx2 = ux2 + co
    ry2 = uy2 + co
    area = (rx2 - rx1) * (ry2 - ry1)
    keep0 = (ts > 0.0).astype(jnp.float32)

    def nms_step(i, keep):
        hm = (cap_iota == i).astype(jnp.float32)
        bx1 = jnp.sum(hm * rx1)
        by1 = jnp.sum(hm * ry1)
        bx2 = jnp.sum(hm * rx2)
        by2 = jnp.sum(hm * ry2)
        ai = jnp.sum(hm * area)
        ki = jnp.sum(hm * keep) > 0.0
        wi = jnp.clip(jnp.minimum(bx2, rx2) - jnp.maximum(bx1, rx1), 0.0, None)
        hi = jnp.clip(jnp.minimum(by2, ry2) - jnp.maximum(by1, ry1), 0.0, None)
        inter = wi * hi
        iou = inter / (ai + area - inter + 1e-9)
        sup = (iou > _NMS_T) & (cap_iota > i) & ki
        return jnp.where(sup, 0.0, keep)

    def nms_body(i, keep):
        keep = nms_step(2 * i, keep)
        keep = nms_step(2 * i + 1, keep)
        return keep

    keep = lax.fori_loop(0, _PRE // 2, nms_body, keep0)
    fs = keep * ts

    row384 = lax.broadcasted_iota(jnp.int32, (_CAP, _CAP), 0)
    col384 = lax.broadcasted_iota(jnp.int32, (_CAP, _CAP), 1)
    ltri = (row384 <= col384).astype(jnp.float32)
    real = (cap_iota < _PRE).astype(jnp.float32)
    nk = (1.0 - keep) * real
    incl_k = jax.lax.dot_general(keep, ltri, (((1,), (0,)), ((), ())),
                                 precision=lax.Precision.HIGHEST,
                                 preferred_element_type=jnp.float32)
    incl_n = jax.lax.dot_general(nk, ltri, (((1,), (0,)), ((), ())),
                                 precision=lax.Precision.HIGHEST,
                                 preferred_element_type=jnp.float32)
    nkept = jnp.sum(keep)
    dest = jnp.where(keep > 0.0, incl_k - 1.0,
                     jnp.where(nk > 0.0, nkept + incl_n - 1.0, 9999.0))
    pt = (row384.astype(jnp.float32) == dest).astype(jnp.float32)
    data = jnp.concatenate(
        [ux1, uy1, ux2, uy2, fs, zrow, zrow, zrow], axis=0)
    outm = jax.lax.dot_general(data, pt, (((1,), (1,)), ((), ())),
                               precision=lax.Precision.HIGHEST,
                               preferred_element_type=jnp.float32)
    out_ref[...] = outm[:, 0:128]


def kernel(boxes, scores, classes, window_idx):
    pad = _NP - _N

    def prep(v):
        return jnp.pad(v, (0, pad)).reshape(_RN, 128)

    x1 = prep(boxes[:, 0])
    y1 = prep(boxes[:, 1])
    x2 = prep(boxes[:, 2])
    y2 = prep(boxes[:, 3])
    s = prep(scores)
    c = prep(classes.astype(jnp.float32))
    w = prep(window_idx.astype(jnp.float32))

    out = pl.pallas_call(
        _nms_kernel,
        out_shape=jax.ShapeDtypeStruct((8, 128), jnp.float32),
        scratch_shapes=[pltpu.VMEM((_RN, 128), jnp.float32)],
    )(x1, y1, x2, y2, s, c, w)

    return jnp.stack([out[0, :_DETS], out[1, :_DETS], out[2, :_DETS],
                      out[3, :_DETS], out[4, :_DETS]], axis=1)


# speedup vs baseline: 1.2105x; 1.0099x over previous
import jax
import jax.numpy as jnp
from jax import lax
from jax.experimental import pallas as pl
from jax.experimental.pallas import tpu as pltpu

_N = 20000
_NP = 20480
_RN = 160
_PRE = 300
_CAP = 384
_DETS = 100
_BORDER = 2.0
_WIN = 1024.0
_IMG = 4096.0
_SAMP = 2.0
_CLS_OFF = 8192.0
_NMS_T = 0.5
_SCORE_T = 1e-8


def _nms_kernel(x1_ref, y1_ref, x2_ref, y2_ref, s_ref, c_ref, w_ref, out_ref,
                s_scr):
    x1 = x1_ref[...]
    y1 = y1_ref[...]
    x2 = x2_ref[...]
    y2 = y2_ref[...]
    s = s_ref[...]
    cls = c_ref[...]

    good = (x1 >= _BORDER) & (y1 >= _BORDER) \
         & (x2 < _WIN - _BORDER) & (y2 < _WIN - _BORDER)
    drone = (cls == 80.0) | (cls == 81.0)
    valid = good & drone & (s > _SCORE_T)
    s_scr[...] = jnp.where(valid, s, 0.0)

    flat = (lax.broadcasted_iota(jnp.int32, (_RN, 128), 0) * 128
            + lax.broadcasted_iota(jnp.int32, (_RN, 128), 1))
    lane = lax.broadcasted_iota(jnp.int32, (1, 128), 1)
    cap_iota = lax.broadcasted_iota(jnp.int32, (1, _CAP), 1)
    zrow = jnp.zeros((1, _CAP), jnp.float32)

    def pick(s_work, slot, carry):
        ts, ux1, uy1, ux2, uy2, uc = carry
        m = jnp.max(s_work)
        idx = jnp.min(jnp.where(s_work == m, flat, jnp.int32(_NP)))
        r = idx // 128
        c = idx - r * 128
        lm = (lane == c).astype(jnp.float32)
        vx1 = jnp.sum(x1_ref[pl.ds(r, 1), :] * lm)
        vy1 = jnp.sum(y1_ref[pl.ds(r, 1), :] * lm)
        vx2 = jnp.sum(x2_ref[pl.ds(r, 1), :] * lm)
        vy2 = jnp.sum(y2_ref[pl.ds(r, 1), :] * lm)
        vc = jnp.sum(c_ref[pl.ds(r, 1), :] * lm)
        vw = jnp.sum(w_ref[pl.ds(r, 1), :] * lm)
        wq = jnp.floor(vw * 0.25)
        offx = wq * 1024.0
        offy = (vw - wq * 4.0) * 1024.0
        hit = (cap_iota == slot).astype(jnp.float32)
        ts = ts + hit * m
        ux1 = ux1 + hit * jnp.clip(vx1 * _SAMP + offx, 0.0, _IMG)
        uy1 = uy1 + hit * jnp.clip(vy1 * _SAMP + offy, 0.0, _IMG)
        ux2 = ux2 + hit * jnp.clip(vx2 * _SAMP + offx, 0.0, _IMG)
        uy2 = uy2 + hit * jnp.clip(vy2 * _SAMP + offy, 0.0, _IMG)
        uc = uc + hit * vc
        s_work = jnp.where(flat == idx, -1.0, s_work)
        return s_work, (ts, ux1, uy1, ux2, uy2, uc)

    def sel_body(i, carry):
        s_work = s_scr[...]
        s_work, carry = pick(s_work, 4 * i, carry)
        s_work, carry = pick(s_work, 4 * i + 1, carry)
        s_work, carry = pick(s_work, 4 * i + 2, carry)
        s_work, carry = pick(s_work, 4 * i + 3, carry)
        s_scr[...] = s_work
        return carry

    carry0 = (zrow, zrow, zrow, zrow, zrow, zrow)
    ts, ux1, uy1, ux2, uy2, uc = lax.fori_loop(0, _PRE // 4, sel_body, carry0)

    co = uc * _CLS_OFF
    rx1 = ux1 + co
    ry1 = uy1 + co
    rx2 = ux2 + co
    ry2 = uy2 + co
    area = (rx2 - rx1) * (ry2 - ry1)
    keep0 = (ts > 0.0).astype(jnp.float32)

    def nms_step(i, keep):
        hm = (cap_iota == i).astype(jnp.float32)
        bx1 = jnp.sum(hm * rx1)
        by1 = jnp.sum(hm * ry1)
        bx2 = jnp.sum(hm * rx2)
        by2 = jnp.sum(hm * ry2)
        ai = jnp.sum(hm * area)
        ki = jnp.sum(hm * keep) > 0.0
        wi = jnp.clip(jnp.minimum(bx2, rx2) - jnp.maximum(bx1, rx1), 0.0, None)
        hi = jnp.clip(jnp.minimum(by2, ry2) - jnp.maximum(by1, ry1), 0.0, None)
        inter = wi * hi
        iou = inter / (ai + area - inter + 1e-9)
        sup = (iou > _NMS_T) & (cap_iota > i) & ki
        return jnp.where(sup, 0.0, keep)

    def nms_body(i, keep):
        keep = nms_step(4 * i, keep)
        keep = nms_step(4 * i + 1, keep)
        keep = nms_step(4 * i + 2, keep)
        keep = nms_step(4 * i + 3, keep)
        return keep

    keep = lax.fori_loop(0, _PRE // 4, nms_body, keep0)
    fs = keep * ts

    row384 = lax.broadcasted_iota(jnp.int32, (_CAP, _CAP), 0)
    col384 = lax.broadcasted_iota(jnp.int32, (_CAP, _CAP), 1)
    ltri = (row384 <= col384).astype(jnp.float32)
    real = (cap_iota < _PRE).astype(jnp.float32)
    nk = (1.0 - keep) * real
    incl_k = jax.lax.dot_general(keep, ltri, (((1,), (0,)), ((), ())),
                                 precision=lax.Precision.HIGHEST,
                                 preferred_element_type=jnp.float32)
    incl_n = jax.lax.dot_general(nk, ltri, (((1,), (0,)), ((), ())),
                                 precision=lax.Precision.HIGHEST,
                                 preferred_element_type=jnp.float32)
    nkept = jnp.sum(keep)
    dest = jnp.where(keep > 0.0, incl_k - 1.0,
                     jnp.where(nk > 0.0, nkept + incl_n - 1.0, 9999.0))
    pt = (row384.astype(jnp.float32) == dest).astype(jnp.float32)
    data = jnp.concatenate(
        [ux1, uy1, ux2, uy2, fs, zrow, zrow, zrow], axis=0)
    outm = jax.lax.dot_general(data, pt, (((1,), (1,)), ((), ())),
                               precision=lax.Precision.HIGHEST,
                               preferred_element_type=jnp.float32)
    out_ref[...] = outm[:, 0:128]


def kernel(boxes, scores, classes, window_idx):
    pad = _NP - _N

    def prep(v):
        return jnp.pad(v, (0, pad)).reshape(_RN, 128)

    x1 = prep(boxes[:, 0])
    y1 = prep(boxes[:, 1])
    x2 = prep(boxes[:, 2])
    y2 = prep(boxes[:, 3])
    s = prep(scores)
    c = prep(classes.astype(jnp.float32))
    w = prep(window_idx.astype(jnp.float32))

    out = pl.pallas_call(
        _nms_kernel,
        out_shape=jax.ShapeDtypeStruct((8, 128), jnp.float32),
        scratch_shapes=[pltpu.VMEM((_RN, 128), jnp.float32)],
    )(x1, y1, x2, y2, s, c, w)

    return jnp.stack([out[0, :_DETS], out[1, :_DETS], out[2, :_DETS],
                      out[3, :_DETS], out[4, :_DETS]], axis=1)
